# Initial kernel scaffold; baseline (speedup 1.0000x reference)
import jax
import jax.numpy as jnp
from jax.experimental import pallas as pl


def kernel(x, edge_index, batch, W1, b1, W2, b2, W3, b3, Wcaps):
    raise NotImplementedError("write your pallas kernel here")


# speedup vs baseline: 1.0650x; 1.0650x over previous
import jax
import jax.numpy as jnp
from jax.experimental import pallas as pl
from jax.experimental.pallas import tpu as pltpu

_B = 100
_C = 96
_OUTLEN = 16
_NC = 10
_ITERS = 3


def _posenc_T(L, d):
    pos = jnp.arange(L, dtype=jnp.float32)[:, None]
    i = jnp.arange(d)[None, :]
    angle = pos / jnp.power(10000.0, (2.0 * (i // 2).astype(jnp.float32)) / float(d))
    pe = jnp.where(i % 2 == 0, jnp.sin(angle), jnp.cos(angle)).astype(jnp.float32)
    return pe.T


def _caps_kernel(scal_ref, xsT_ref, peT_ref, wc_ref, out_ref):
    b = pl.program_id(0)
    off = pl.multiple_of(scal_ref[0, b], 128)
    cnt = scal_ref[1, b]
    mk = scal_ref[2, 0]
    K = peT_ref.shape[1]
    XT = xsT_ref[:, pl.ds(off, K)]
    n = jax.lax.broadcasted_iota(jnp.int32, (1, K), 1)
    node_m = n < cnt
    valid_m = n < mk
    denseT = jnp.where(node_m, XT, 0.0)
    denseT = jnp.where(valid_m, denseT + peT_ref[...], 0.0)
    denom = mk.astype(jnp.float32)
    vals = []
    for oc in range(_NC):
        W = wc_ref[oc]
        P = jnp.dot(W, denseT, preferred_element_type=jnp.float32)
        out = jnp.sum(P, axis=1, keepdims=True) / denom
        for _ in range(_ITERS):
            nrm = jnp.sqrt(jnp.sum(out * out))
            out_n = out / jnp.maximum(nrm, 1e-12)
            logits = jnp.sum(P * out_n, axis=0, keepdims=True)
            mmax = jnp.max(jnp.where(valid_m, logits, -1e30))
            unnorm = jnp.where(valid_m, jnp.exp(logits - mmax), 0.0)
            probs = unnorm / jnp.sum(unnorm)
            out = jnp.sum(probs * P, axis=1, keepdims=True)
        vals.append(jnp.sqrt(jnp.sum(out * out)).reshape(1, 1))
    vals.append(jnp.zeros((1, 128 - _NC), jnp.float32))
    out_ref[pl.ds(b, 1), :] = jnp.concatenate(vals, axis=1)


def kernel(x, edge_index, batch, W1, b1, W2, b2, W3, b3, Wcaps):
    N = x.shape[0]
    src, dst = edge_index[0], edge_index[1]
    mask = (src != dst).astype(jnp.float32)
    deg = jnp.zeros((N,), jnp.float32).at[dst].add(mask) + 1.0
    dinv = 1.0 / jnp.sqrt(deg)
    norm = dinv[src] * dinv[dst] * mask

    def gcn(h_in, W, b):
        h = h_in @ W
        agg = jnp.zeros((N, h.shape[1]), jnp.float32).at[dst].add(norm[:, None] * h[src])
        agg = agg + (dinv * dinv)[:, None] * h
        return jnp.tanh(agg + b)

    x1 = gcn(x, W1, b1)
    x2 = gcn(x1, W2, b2)
    x3 = gcn(x2, W3, b3)
    xc = jnp.concatenate([x1, x2, x3], axis=-1)

    counts = jnp.bincount(batch, length=_B)
    maxk = jnp.max(counts)
    offsets = jnp.cumsum(counts) - counts
    order = jnp.lexsort((-xc[:, -1], batch))
    bs = batch[order]
    xs = xc[order]
    posn = jnp.arange(N) - offsets[bs]

    ALIGN = 128
    KCAP = 10240
    TOT = 33024
    acounts = ((counts + ALIGN - 1) // ALIGN) * ALIGN
    aoffs = jnp.cumsum(acounts) - acounts
    p = aoffs[bs] + posn
    xsal = jnp.zeros((TOT, _C), jnp.float32).at[p].set(xs)
    xsT_pad = xsal.T
    peT = jnp.zeros((_C, KCAP), jnp.float32).at[:, :N].set(_posenc_T(N, _C))
    scal = jnp.stack([
        aoffs.astype(jnp.int32),
        counts.astype(jnp.int32),
        jnp.full((_B,), maxk, jnp.int32),
    ], axis=0)
    wc = Wcaps.astype(jnp.float32)

    grid_spec = pltpu.PrefetchScalarGridSpec(
        num_scalar_prefetch=1,
        grid=(_B,),
        in_specs=[
            pl.BlockSpec(xsT_pad.shape, lambda b, s: (0, 0)),
            pl.BlockSpec(peT.shape, lambda b, s: (0, 0)),
            pl.BlockSpec(wc.shape, lambda b, s: (0, 0, 0)),
        ],
        out_specs=pl.BlockSpec((_B, 128), lambda b, s: (0, 0)),
    )
    classes = pl.pallas_call(
        _caps_kernel,
        grid_spec=grid_spec,
        out_shape=jax.ShapeDtypeStruct((_B, 128), jnp.float32),
    )(scal, xsT_pad, peT, wc)
    return classes[:, :_NC]
